# baseline (speedup 1.0000x reference)
import jax
import jax.numpy as jnp
from jax.experimental import pallas as pl
from jax.experimental.pallas import tpu as pltpu

CAP = 100000
D = 256
VDIM = 64
K = 5
BLK = 16384
NB = (CAP + BLK - 1) // BLK
G = BLK // 128

NEG = float("-inf")
IMAX = 2**31 - 1


def _top5_lanes(sim, idx, nlane):
    lane = jax.lax.broadcasted_iota(jnp.int32, (1, nlane), 1)
    v_out = jnp.full((1, nlane), NEG, dtype=jnp.float32)
    i_out = jnp.zeros((1, nlane), dtype=jnp.int32)
    for j in range(K):
        m = jnp.max(sim)
        sel = jnp.min(jnp.where(sim == m, idx, IMAX))
        v_out = jnp.where(lane == j, m, v_out)
        i_out = jnp.where(lane == j, sel, i_out)
        sim = jnp.where((sim == m) & (idx == sel), NEG, sim)
    return v_out, i_out


def _fused_kernel(query_ref, W1_ref, b1_ref, W2_ref, b2_ref, gamma_ref,
                  beta_ref, keys_ref, values_ref, out_ref,
                  qn_ref, rv_ref, ri_ref, rows_ref, idx_smem, sem):
    i = pl.program_id(0)

    @pl.when(i == 0)
    def _():
        q = query_ref[...]
        h = jnp.dot(q, W1_ref[...], preferred_element_type=jnp.float32) + b1_ref[...]
        h = h * jax.nn.sigmoid(h)
        h = jnp.dot(h, W2_ref[...], preferred_element_type=jnp.float32) + b2_ref[...]
        mean = jnp.mean(h, axis=-1, keepdims=True)
        var = jnp.mean((h - mean) * (h - mean), axis=-1, keepdims=True)
        h = (h - mean) * jax.lax.rsqrt(var + 1e-5) * gamma_ref[...] + beta_ref[...]
        n = jnp.sqrt(jnp.sum(h * h))
        qn_ref[...] = h / jnp.maximum(n, 1e-12)
        rv_ref[...] = jnp.full((1, 128), NEG, dtype=jnp.float32)
        ri_ref[...] = jnp.zeros((1, 128), dtype=jnp.int32)

    x3 = keys_ref[...].reshape(G, 128, D)
    qn = qn_ref[...].reshape(1, 1, D)
    dot = jnp.sum(x3 * qn, axis=2)
    ss = jnp.sum(x3 * x3, axis=2)
    sim = dot / jnp.maximum(jnp.sqrt(ss), 1e-12)

    iota = (jax.lax.broadcasted_iota(jnp.int32, (G, 128), 0) * 128
            + jax.lax.broadcasted_iota(jnp.int32, (G, 128), 1) + i * BLK)
    sim = jnp.where(iota < CAP, sim, NEG)
    bv, bi = _top5_lanes(sim, iota, 128)

    cat_v = jnp.concatenate([rv_ref[...], bv], axis=0)
    cat_i = jnp.concatenate([ri_ref[...], bi], axis=0)
    nv, ni = _top5_lanes(cat_v, cat_i, 128)
    rv_ref[...] = nv
    ri_ref[...] = ni

    @pl.when(i == NB - 1)
    def _():
        lane = jax.lax.broadcasted_iota(jnp.int32, (1, 128), 1)
        fv = rv_ref[...]
        fi = ri_ref[...]
        svals = [jnp.sum(jnp.where(lane == j, fv, 0.0)) for j in range(K)]
        for j in range(K):
            idx_smem[j] = jnp.sum(jnp.where(lane == j, fi, 0))
        for j in range(K):
            pltpu.make_async_copy(
                values_ref.at[pl.ds(idx_smem[j], 1), :],
                rows_ref.at[pl.ds(j, 1), :],
                sem,
            ).start()
        m0 = svals[0]
        es = [jnp.exp(s - m0) for s in svals]
        denom = es[0] + es[1] + es[2] + es[3] + es[4]
        for j in range(K):
            pltpu.make_async_copy(
                values_ref.at[pl.ds(idx_smem[j], 1), :],
                rows_ref.at[pl.ds(j, 1), :],
                sem,
            ).wait()
        rows = rows_ref[...]
        acc = (es[0] / denom) * rows[0:1, :]
        for j in range(1, K):
            acc = acc + (es[j] / denom) * rows[j:j + 1, :]
        out_ref[...] = acc


@jax.jit
def kernel(query, keys, values, W1, b1, W2, b2, gamma, beta):
    b1r = b1.reshape(1, D)
    b2r = b2.reshape(1, D)
    gr = gamma.reshape(1, D)
    br = beta.reshape(1, D)

    out = pl.pallas_call(
        _fused_kernel,
        grid=(NB,),
        in_specs=[
            pl.BlockSpec((1, D), lambda i: (0, 0)),
            pl.BlockSpec((D, D), lambda i: (0, 0)),
            pl.BlockSpec((1, D), lambda i: (0, 0)),
            pl.BlockSpec((D, D), lambda i: (0, 0)),
            pl.BlockSpec((1, D), lambda i: (0, 0)),
            pl.BlockSpec((1, D), lambda i: (0, 0)),
            pl.BlockSpec((1, D), lambda i: (0, 0)),
            pl.BlockSpec((BLK, D), lambda i: (i, 0)),
            pl.BlockSpec(memory_space=pl.ANY),
        ],
        out_specs=pl.BlockSpec((1, VDIM), lambda i: (0, 0)),
        out_shape=jax.ShapeDtypeStruct((1, VDIM), jnp.float32),
        scratch_shapes=[
            pltpu.VMEM((1, D), jnp.float32),
            pltpu.VMEM((1, 128), jnp.float32),
            pltpu.VMEM((1, 128), jnp.int32),
            pltpu.VMEM((8, VDIM), jnp.float32),
            pltpu.SMEM((8,), jnp.int32),
            pltpu.SemaphoreType.DMA,
        ],
    )(query, W1, b1r, W2, b2r, gr, br, keys, values)

    return out.reshape(VDIM)


# speedup vs baseline: 1.0902x; 1.0902x over previous
import jax
import jax.numpy as jnp
from jax.experimental import pallas as pl
from jax.experimental.pallas import tpu as pltpu

CAP = 100000
D = 256
VDIM = 64
K = 5
BLK = 16384
NB = (CAP + BLK - 1) // BLK
G = BLK // 128

NEG = float("-inf")
IMAX = 2**31 - 1


def _top5_lanes(sim, idx, nlane):
    lane = jax.lax.broadcasted_iota(jnp.int32, (1, nlane), 1)
    v_out = jnp.full((1, nlane), NEG, dtype=jnp.float32)
    i_out = jnp.zeros((1, nlane), dtype=jnp.int32)
    for j in range(K):
        m = jnp.max(sim)
        sel = jnp.min(jnp.where(sim == m, idx, IMAX))
        v_out = jnp.where(lane == j, m, v_out)
        i_out = jnp.where(lane == j, sel, i_out)
        sim = jnp.where((sim == m) & (idx == sel), NEG, sim)
    return v_out, i_out


def _fused_kernel(query_ref, W1_ref, b1_ref, W2_ref, b2_ref, gamma_ref,
                  beta_ref, keys_ref, values_ref, out_ref,
                  qn_ref, cv_ref, ci_ref, rows_ref, idx_smem, sem):
    i = pl.program_id(0)

    @pl.when(i == 0)
    def _():
        q = query_ref[...]
        h = jnp.dot(q, W1_ref[...], preferred_element_type=jnp.float32) + b1_ref[...]
        h = h * jax.nn.sigmoid(h)
        h = jnp.dot(h, W2_ref[...], preferred_element_type=jnp.float32) + b2_ref[...]
        mean = jnp.mean(h, axis=-1, keepdims=True)
        var = jnp.mean((h - mean) * (h - mean), axis=-1, keepdims=True)
        h = (h - mean) * jax.lax.rsqrt(var + 1e-5) * gamma_ref[...] + beta_ref[...]
        n = jnp.sqrt(jnp.sum(h * h))
        qn_ref[...] = h / jnp.maximum(n, 1e-12)
        cv_ref[...] = jnp.full((8, 128), NEG, dtype=jnp.float32)
        ci_ref[...] = jnp.zeros((8, 128), dtype=jnp.int32)

    x3 = keys_ref[...].reshape(G, 128, D)
    qn = qn_ref[...].reshape(1, 1, D)
    dot = jnp.sum(x3 * qn, axis=2)
    ss = jnp.sum(x3 * x3, axis=2)
    sim = dot / jnp.maximum(jnp.sqrt(ss), 1e-12)

    iota = (jax.lax.broadcasted_iota(jnp.int32, (G, 128), 0) * 128
            + jax.lax.broadcasted_iota(jnp.int32, (G, 128), 1) + i * BLK)
    sim = jnp.where(iota < CAP, sim, NEG)
    bv, bi = _top5_lanes(sim, iota, 128)
    cv_ref[pl.ds(i, 1), :] = bv
    ci_ref[pl.ds(i, 1), :] = bi

    @pl.when(i == NB - 1)
    def _():
        lane = jax.lax.broadcasted_iota(jnp.int32, (1, 128), 1)
        fv, fi = _top5_lanes(cv_ref[...], ci_ref[...], 128)
        svals = [jnp.sum(jnp.where(lane == j, fv, 0.0)) for j in range(K)]
        for j in range(K):
            idx_smem[j] = jnp.sum(jnp.where(lane == j, fi, 0))
        for j in range(K):
            pltpu.make_async_copy(
                values_ref.at[pl.ds(idx_smem[j], 1), :],
                rows_ref.at[pl.ds(j, 1), :],
                sem,
            ).start()
        m0 = svals[0]
        es = [jnp.exp(s - m0) for s in svals]
        denom = es[0] + es[1] + es[2] + es[3] + es[4]
        for j in range(K):
            pltpu.make_async_copy(
                values_ref.at[pl.ds(idx_smem[j], 1), :],
                rows_ref.at[pl.ds(j, 1), :],
                sem,
            ).wait()
        rows = rows_ref[...]
        acc = (es[0] / denom) * rows[0:1, :]
        for j in range(1, K):
            acc = acc + (es[j] / denom) * rows[j:j + 1, :]
        out_ref[...] = acc


@jax.jit
def kernel(query, keys, values, W1, b1, W2, b2, gamma, beta):
    b1r = b1.reshape(1, D)
    b2r = b2.reshape(1, D)
    gr = gamma.reshape(1, D)
    br = beta.reshape(1, D)

    out = pl.pallas_call(
        _fused_kernel,
        grid=(NB,),
        in_specs=[
            pl.BlockSpec((1, D), lambda i: (0, 0)),
            pl.BlockSpec((D, D), lambda i: (0, 0)),
            pl.BlockSpec((1, D), lambda i: (0, 0)),
            pl.BlockSpec((D, D), lambda i: (0, 0)),
            pl.BlockSpec((1, D), lambda i: (0, 0)),
            pl.BlockSpec((1, D), lambda i: (0, 0)),
            pl.BlockSpec((1, D), lambda i: (0, 0)),
            pl.BlockSpec((BLK, D), lambda i: (i, 0)),
            pl.BlockSpec(memory_space=pl.ANY),
        ],
        out_specs=pl.BlockSpec((1, VDIM), lambda i: (0, 0)),
        out_shape=jax.ShapeDtypeStruct((1, VDIM), jnp.float32),
        scratch_shapes=[
            pltpu.VMEM((1, D), jnp.float32),
            pltpu.VMEM((8, 128), jnp.float32),
            pltpu.VMEM((8, 128), jnp.int32),
            pltpu.VMEM((8, VDIM), jnp.float32),
            pltpu.SMEM((8,), jnp.int32),
            pltpu.SemaphoreType.DMA,
        ],
    )(query, W1, b1r, W2, b2r, gr, br, keys, values)

    return out.reshape(VDIM)
